# baseline (speedup 1.0000x reference)
import functools

import jax
import jax.numpy as jnp
from jax import lax
from jax.experimental import pallas as pl
from jax.experimental.pallas import tpu as pltpu
from jax.experimental.pallas import tpu_sc as plsc

N_NODES = 10000
NACC = 10240
SPLIT = 5120
HALF = 5376
DUMP = 5120
K = 128
NCH_MAX = 46
ECAP = NCH_MAX * 16 * K
G = 256
CB = 128


def _pad2(a, r, c):
    return jnp.pad(a, ((0, r - a.shape[0]), (0, c - a.shape[1])))


def _pad1(a, n):
    return jnp.pad(a, (0, n - a.shape[0]))


def _sc_gat_body(ncb, h_hbm, asrc_hbm, adst_hbm, s_hbm, d_hbm, dg_hbm,
                 out_hbm, ab_v, s_all, dloc_all, dglob_all, e_all, w_all,
                 w_v, den_g, rows2_v, wbuf_v, zden_v, den_sp, out_sp,
                 sg0, sg1, ss, sd):
    c = lax.axis_index("c")
    t = lax.axis_index("s")
    sg = (sg0, sg1)

    pltpu.sync_copy(s_hbm.at[c].at[t], s_all)
    pltpu.sync_copy(d_hbm.at[c].at[t], dloc_all)
    pltpu.sync_copy(dg_hbm.at[c].at[t], dglob_all)

    def _zrow(r, _):
        for u in range(CB // 16):
            wbuf_v[r, pl.ds(u * 16, 16)] = jnp.zeros((16,), jnp.float32)
        return 0
    lax.fori_loop(0, 48, _zrow, 0)
    def _zden(i, _):
        zden_v[pl.ds(i * 16, 16)] = jnp.zeros((16,), jnp.float32)
        return 0
    lax.fori_loop(0, 21, _zden, 0)
    pltpu.sync_copy(zden_v, den_sp.at[pl.ds(t * 336, 336)])
    plsc.subcore_barrier()

    def _fire_g1(j, slot):
        pltpu.async_copy(asrc_hbm.at[s_all.at[j]], ab_v.at[slot, 0],
                         sg[slot])
        pltpu.async_copy(adst_hbm.at[dglob_all.at[j]],
                         ab_v.at[slot, 1], sg[slot])

    _fire_g1(0, 0)

    def _p1(jj, _):
        for hh in range(2):
            j = jj * 2 + hh
            if hh == 0:
                _fire_g1(j + 1, 1)
            else:
                @pl.when(jj < NCH_MAX // 2 - 1)
                def _():
                    _fire_g1(j + 1, 0)
            pltpu.make_async_copy(asrc_hbm.at[s_all.at[j]],
                                  ab_v.at[hh, 0], sg[hh]).wait()
            pltpu.make_async_copy(adst_hbm.at[dglob_all.at[j]],
                                  ab_v.at[hh, 1], sg[hh]).wait()
            for u in range(K // 16):
                sl = pl.ds(u * 16, 16)
                a = ab_v[hh, 0, sl] + ab_v[hh, 1, sl]
                a = jnp.where(a >= 0.0, a, a * 0.2)
                e_all[j, sl] = jnp.exp(a)
            if hh == 0:
                @pl.when(jj > 0)
                def _():
                    pltpu.make_async_copy(
                        e_all.at[j], den_sp.at[dloc_all.at[j]], ss).wait()
            else:
                pltpu.make_async_copy(
                    e_all.at[j], den_sp.at[dloc_all.at[j]], ss).wait()
            pltpu.async_copy(e_all.at[j], den_sp.at[dloc_all.at[j]], ss,
                             add=True)
        return 0
    lax.fori_loop(0, NCH_MAX // 2, _p1, 0)
    pltpu.make_async_copy(e_all.at[0], den_sp.at[dloc_all.at[0]], ss).wait()
    plsc.subcore_barrier()

    for cb in range(ncb):
        for q in range(7):
            pltpu.sync_copy(wbuf_v, out_sp.at[pl.ds(t * 336 + q * 48, 48)])
        plsc.subcore_barrier()

        def _fire_g2(j, slot, cb=cb):
            pltpu.async_copy(h_hbm.at[cb].at[s_all.at[j]],
                             rows2_v.at[slot], sg[slot])

        _fire_g2(0, 0)

        def _p2(jj, _, cb=cb):
            for hh in range(2):
                j = jj * 2 + hh
                if hh == 0:
                    @pl.when(jj > 0)
                    def _():
                        pltpu.make_async_copy(
                            rows2_v.at[1], out_sp.at[dloc_all.at[j]],
                            ss).wait()
                    _fire_g2(j + 1, 1)
                else:
                    pltpu.make_async_copy(
                        rows2_v.at[0], out_sp.at[dloc_all.at[j]], ss).wait()
                    @pl.when(jj < NCH_MAX // 2 - 1)
                    def _():
                        _fire_g2(j + 1, 0)
                pltpu.make_async_copy(h_hbm.at[cb].at[s_all.at[j]],
                                      rows2_v.at[hh], sg[hh]).wait()
                if cb == 0:
                    pltpu.async_copy(den_sp.at[dloc_all.at[j]], den_g,
                                     sd).wait()
                for u in range(K // 16):
                    sl = pl.ds(u * 16, 16)
                    if cb == 0:
                        w_all[j, sl] = e_all[j, sl] / (den_g[sl] + 1e-16)
                    w_v[sl] = w_all[j, sl]
                def _scale(k2, _):
                    for k in (k2 * 2, k2 * 2 + 1):
                        wv = w_v[pl.ds((k // 16) * 16, 16)]
                        wk = jnp.take_along_axis(
                            wv, jnp.full((16,), k % 16, jnp.int32), axis=0)
                        for u in range(CB // 16):
                            sl = pl.ds(u * 16, 16)
                            rows2_v[hh, k, sl] = rows2_v[hh, k, sl] * wk
                    return 0
                lax.fori_loop(0, K // 2, _scale, 0)
                pltpu.async_copy(rows2_v.at[hh], out_sp.at[dloc_all.at[j]],
                                 ss, add=True)
            return 0
        lax.fori_loop(0, NCH_MAX // 2, _p2, 0)
        pltpu.make_async_copy(rows2_v.at[1], out_sp.at[dloc_all.at[0]],
                              ss).wait()
        plsc.subcore_barrier()

        for q in range(7):
            r0 = t * 336 + q * 48
            pltpu.sync_copy(out_sp.at[pl.ds(r0, 48)],
                            rows2_v.at[0].at[pl.ds(0, 48)])
            pltpu.sync_copy(rows2_v.at[0].at[pl.ds(0, 48)],
                            out_hbm.at[c].at[cb].at[pl.ds(r0, 48)])
        plsc.subcore_barrier()


@functools.partial(jax.jit, static_argnums=(0,))
def _sc_gat(ncb, h, asrc, adst, s2, d2, d2g):
    mesh = plsc.VectorSubcoreMesh(core_axis_name="c", subcore_axis_name="s",
                                  num_cores=2, num_subcores=16)
    kfn = pl.kernel(
        functools.partial(_sc_gat_body, ncb),
        out_type=jax.ShapeDtypeStruct((2, ncb, HALF, CB), jnp.float32),
        mesh=mesh,
        scratch_types=[
            pltpu.VMEM((2, 2, K), jnp.float32),
            pltpu.VMEM((NCH_MAX, K), jnp.int32),
            pltpu.VMEM((NCH_MAX, K), jnp.int32),
            pltpu.VMEM((NCH_MAX, K), jnp.int32),
            pltpu.VMEM((NCH_MAX, K), jnp.float32),
            pltpu.VMEM((NCH_MAX, K), jnp.float32),
            pltpu.VMEM((K,), jnp.float32),
            pltpu.VMEM((K,), jnp.float32),
            pltpu.VMEM((2, K, CB), jnp.float32),
            pltpu.VMEM((48, CB), jnp.float32),
            pltpu.VMEM((336,), jnp.float32),
            pltpu.VMEM_SHARED((HALF,), jnp.float32),
            pltpu.VMEM_SHARED((HALF, CB), jnp.float32),
            pltpu.SemaphoreType.DMA,
            pltpu.SemaphoreType.DMA,
            pltpu.SemaphoreType.DMA,
            pltpu.SemaphoreType.DMA,
        ],
        name=f"sc_gat_{ncb}",
    )
    return kfn(h, asrc, adst, s2, d2, d2g)


RB = 1024


def _rows_of(p_refs, b_ref):
    z = jnp.concatenate([p[0, 0] for p in p_refs], axis=1)
    return jnp.maximum(z + b_ref[...], 0.0)


def _tc_layer_body(variant, nin, x_refs, h_ref, av_ref):
    cb = pl.program_id(1)
    if variant == "x":
        xin = x_refs[0][...]
        rest = x_refs[1:]
    elif variant == "p":
        xin = _rows_of(x_refs[:nin], x_refs[nin])
        rest = x_refs[nin + 1:]
    else:
        z = _rows_of(x_refs[:nin], x_refs[nin])
        supp_ref, ids_ref = x_refs[nin + 1], x_refs[nin + 2]
        ids = ids_ref[0]
        oh = (lax.broadcasted_iota(jnp.int32, (G, RB), 0) == ids).astype(
            jnp.float32)
        sy = lax.dot_general(oh, supp_ref[...], (((0,), (0,)), ((), ())),
                             preferred_element_type=jnp.float32)
        xin = jnp.concatenate([z + sy, z - sy], axis=1)
        rest = x_refs[nin + 3:]
    w_ref, a8_ref = rest
    h = jnp.dot(xin, w_ref[...], preferred_element_type=jnp.float32)
    h_ref[0] = h
    @pl.when(cb == 0)
    def _():
        av_ref[...] = jnp.zeros_like(av_ref)
    av_ref[...] += lax.dot_general(a8_ref[...], h, (((1,), (1,)), ((), ())),
                                   preferred_element_type=jnp.float32)


def _full(shape):
    return pl.BlockSpec(shape, lambda r, cb: tuple(0 for _ in shape))


def _p_specs(nin):
    return [pl.BlockSpec((1, 1, RB, CB),
                         lambda r, cb, i=i: (r // 5, i, r % 5, 0))
            for i in range(nin)]


def _tc_layer(variant, fin, fout, inputs):
    ncb = fout // CB
    if variant == "x":
        nin = 0
        in_specs = [pl.BlockSpec((RB, fin), lambda r, cb: (r, 0))]
    elif variant == "p":
        nin = fin // CB
        in_specs = _p_specs(nin) + [_full((1, fin))]
    else:
        fs = fin // 2
        nin = fs // CB
        in_specs = _p_specs(nin) + [_full((1, fs)), _full((G, fs)),
                                    pl.BlockSpec((1, 1, RB),
                                                 lambda r, cb: (r, 0, 0))]
    in_specs += [pl.BlockSpec((fin, CB), lambda r, cb: (0, cb)),
                 pl.BlockSpec((8, CB), lambda r, cb: (0, cb))]
    return pl.pallas_call(
        lambda *refs: _tc_layer_body(variant, nin, refs[:-2], refs[-2],
                                     refs[-1]),
        grid=(NACC // RB, ncb),
        in_specs=in_specs,
        out_specs=[pl.BlockSpec((1, RB, CB), lambda r, cb: (cb, r, 0)),
                   pl.BlockSpec((8, RB), lambda r, cb: (0, r))],
        out_shape=[jax.ShapeDtypeStruct((ncb, NACC, CB), jnp.float32),
                   jax.ShapeDtypeStruct((8, NACC), jnp.float32)],
    )(*inputs)


def _pool_body(nin, refs):
    p_refs = refs[:nin]
    b_ref, ids_ref, pooled_ref, cnt_ref = refs[nin:]
    r = pl.program_id(0)
    z = _rows_of(p_refs, b_ref)
    ids = ids_ref[0]
    oh = (lax.broadcasted_iota(jnp.int32, (G, RB), 0) == ids).astype(
        jnp.float32)
    ps = lax.dot_general(oh, z, (((1,), (0,)), ((), ())),
                         preferred_element_type=jnp.float32)
    cs = jnp.sum(oh, axis=1, keepdims=True) * jnp.ones((1, 8), jnp.float32)
    @pl.when(r == 0)
    def _():
        pooled_ref[...] = jnp.zeros_like(pooled_ref)
        cnt_ref[...] = jnp.zeros_like(cnt_ref)
    pooled_ref[...] += ps
    cnt_ref[...] += cs


def _pool_p_specs(nin):
    return [pl.BlockSpec((1, 1, RB, CB),
                         lambda r, i=i: (r // 5, i, r % 5, 0))
            for i in range(nin)]


def _tc_pool(fp, p, b, ids3):
    nin = fp // CB
    return pl.pallas_call(
        lambda *refs: _pool_body(nin, refs),
        grid=(NACC // RB,),
        in_specs=_pool_p_specs(nin) + [
            pl.BlockSpec((1, fp), lambda r: (0, 0)),
            pl.BlockSpec((1, 1, RB), lambda r: (r, 0, 0))],
        out_specs=[pl.BlockSpec((G, fp), lambda r: (0, 0)),
                   pl.BlockSpec((G, 8), lambda r: (0, 0))],
        out_shape=[jax.ShapeDtypeStruct((G, fp), jnp.float32),
                   jax.ShapeDtypeStruct((G, 8), jnp.float32)],
    )(*([p] * nin + [b, ids3]))


def _mlp_body(x_ref, w1_ref, b1_ref, w2_ref, b2_ref, o_ref):
    h = jnp.maximum(jnp.dot(x_ref[...], w1_ref[...],
                            preferred_element_type=jnp.float32)
                    + b1_ref[...], 0.0)
    o_ref[...] = jnp.dot(h, w2_ref[...],
                         preferred_element_type=jnp.float32) + b2_ref[...]


def _tc_mlp(x, w1, b1, w2, b2):
    return pl.pallas_call(
        _mlp_body,
        out_shape=jax.ShapeDtypeStruct((x.shape[0], w2.shape[1]),
                                       jnp.float32),
    )(x, w1, b1, w2, b2)


def _mean_mlp_body(x_ref, cnt_ref, w1_ref, b1_ref, w2_ref, b2_ref, o_ref):
    cnt = jnp.maximum(cnt_ref[:, 0:1], 1.0)
    x = x_ref[...] / cnt
    h = jnp.maximum(jnp.dot(x, w1_ref[...],
                            preferred_element_type=jnp.float32)
                    + b1_ref[...], 0.0)
    o_ref[...] = jnp.dot(h, w2_ref[...],
                         preferred_element_type=jnp.float32) + b2_ref[...]


def _tc_mean_mlp(pooled, cnt, w1, b1, w2, b2):
    return pl.pallas_call(
        _mean_mlp_body,
        out_shape=jax.ShapeDtypeStruct((G, w2.shape[1]), jnp.float32),
    )(pooled, cnt, w1, b1, w2, b2)


def _cnn_body(ids_ref, emb_ref, w1_ref, b1_ref, w2_ref, b2_ref, o_ref):
    ids = ids_ref[0]
    oh = (lax.broadcasted_iota(jnp.int32, (65, 128), 0) == ids).astype(
        jnp.float32)
    xe = lax.dot_general(oh, emb_ref[...], (((0,), (0,)), ((), ())),
                         preferred_element_type=jnp.float32)
    z1 = jnp.zeros((1, 64), jnp.float32)
    xp = jnp.concatenate([z1, xe[0:100], z1], axis=0)
    y1 = jnp.dot(xp[0:99], w1_ref[0], preferred_element_type=jnp.float32)
    for k2 in range(1, 4):
        y1 += jnp.dot(xp[k2:k2 + 99], w1_ref[k2],
                      preferred_element_type=jnp.float32)
    y1 = jnp.maximum(y1 + b1_ref[...], 0.0)
    z2 = jnp.zeros((1, 512), jnp.float32)
    y1p = jnp.concatenate([z2, y1, z2], axis=0)
    y2 = jnp.dot(y1p[0:98], w2_ref[0], preferred_element_type=jnp.float32)
    for k2 in range(1, 4):
        y2 += jnp.dot(y1p[k2:k2 + 98], w2_ref[k2],
                      preferred_element_type=jnp.float32)
    y2 = jnp.maximum(y2 + b2_ref[...], 0.0)
    o_ref[...] = jnp.max(y2[0:97], axis=0, keepdims=True)[None]


def _tc_cnn(ids3, emb, w1r, b1, w2r, b2):
    return pl.pallas_call(
        _cnn_body,
        grid=(G,),
        in_specs=[pl.BlockSpec((1, 1, 128), lambda g: (g, 0, 0)),
                  pl.BlockSpec((65, 64), lambda g: (0, 0)),
                  pl.BlockSpec((4, 64, 512), lambda g: (0, 0, 0)),
                  pl.BlockSpec((1, 512), lambda g: (0, 0)),
                  pl.BlockSpec((4, 512, 256), lambda g: (0, 0, 0)),
                  pl.BlockSpec((1, 256), lambda g: (0, 0))],
        out_specs=pl.BlockSpec((1, 1, 256), lambda g: (g, 0, 0)),
        out_shape=jax.ShapeDtypeStruct((G, 1, 256), jnp.float32),
    )(ids3, emb, w1r, b1, w2r, b2).reshape(G, 256)


def _edge_setup(edge_index):
    src = edge_index[0].astype(jnp.int32)
    dst = edge_index[1].astype(jnp.int32)
    loop = jnp.arange(N_NODES, dtype=jnp.int32)
    s_all = jnp.concatenate([src, loop])
    d_all = jnp.concatenate([dst, loop])
    half = (d_all >= SPLIT).astype(jnp.int32)
    d_loc = d_all - SPLIT * half
    r1 = jnp.cumsum(half) - half
    r0 = jnp.cumsum(1 - half) - (1 - half)
    pos = jnp.where(half, ECAP + r1, r0)
    ar = jnp.arange(2 * ECAP, dtype=jnp.int32)
    s_buf = 10016 + (ar % 224)
    d_buf = jnp.where(ar < ECAP, DUMP + (ar % 256), 4880 + (ar % 240))
    dg_buf = jnp.where(ar < ECAP, DUMP + (ar % 256), N_NODES + (ar % 240))
    s_buf = s_buf.at[pos].set(s_all)
    d_buf = d_buf.at[pos].set(d_loc)
    dg_buf = dg_buf.at[pos].set(d_all)

    def _lay(b):
        return b.reshape(2, NCH_MAX, 16, K).transpose(0, 2, 1, 3)
    return _lay(s_buf), _lay(d_buf), _lay(dg_buf)


def _arrange_w2(w, fs, fsp, fop):
    out = jnp.zeros((2 * fsp, fop), jnp.float32)
    out = out.at[0:fs, 0:w.shape[1]].set(w[0:fs])
    out = out.at[fsp:fsp + fs, 0:w.shape[1]].set(w[fs:2 * fs])
    return out


def _a8(p, fop):
    a = jnp.zeros((8, fop), jnp.float32)
    a = a.at[0, 0:p["a_src"].shape[0]].set(p["a_src"])
    a = a.at[1, 0:p["a_dst"].shape[0]].set(p["a_dst"])
    return a


def _gat_stack(x, edge_index, batch, supp, layers, dims, out_mlp):
    s2, d2, d2g = _edge_setup(edge_index)
    bat3 = jnp.concatenate(
        [batch.astype(jnp.int32),
         jnp.full((NACC - N_NODES,), G, jnp.int32)]).reshape(NACC // RB, 1, RB)

    fin0 = dims[0][0]
    x_p = _pad2(x, NACC, fin0)
    p_half = None
    for i, (fin, fout) in enumerate(dims):
        pp = layers[i]
        w = _pad2(pp["W"], fin, fout)
        if i == 2:
            fs = layers[1]["W"].shape[1]
            w = _arrange_w2(pp["W"], fs, fin // 2, fout)
        a8 = _a8(pp, fout)
        if i == 0:
            h, av = _tc_layer("x", fin, fout, [x_p, w, a8])
        elif i == 2:
            bprev = _pad1(layers[1]["b"], fin // 2).reshape(1, fin // 2)
            h, av = _tc_layer("ps", fin, fout,
                              [p_half] * (fin // 2 // CB)
                              + [bprev, supp, bat3, w, a8])
        else:
            bprev = _pad1(layers[i - 1]["b"], fin).reshape(1, fin)
            h, av = _tc_layer("p", fin, fout,
                              [p_half] * (fin // CB) + [bprev, w, a8])
        p_half = _sc_gat(fout // CB, h, av[0], av[1], s2, d2, d2g)

    fp = dims[-1][1]
    b_last = _pad1(layers[-1]["b"], fp).reshape(1, fp)
    pooled, cnt = _tc_pool(fp, p_half, b_last, bat3)
    w1 = _pad2(out_mlp["l1"]["W"], fp, 1024)
    return _tc_mean_mlp(pooled, cnt, w1,
                        out_mlp["l1"]["b"].reshape(1, 1024),
                        out_mlp["l2"]["W"],
                        out_mlp["l2"]["b"].reshape(1, 128))


def kernel(drug_x, drug_edge_index, drug_batch, drug_smiles, target_x,
           target_edge_index, target_batch, target_esm2, params):
    cnn = params["cnn"]
    ids3 = jnp.pad(drug_smiles.astype(jnp.int32),
                   ((0, 0), (0, 28))).reshape(G, 1, 128)
    w1r = jnp.transpose(cnn["w1"], (2, 1, 0))
    w2r = jnp.pad(jnp.transpose(cnn["w2"], (2, 1, 0)),
                  ((0, 0), (0, 0), (0, 100)))
    drug_supp = _tc_cnn(ids3, cnn["emb"], w1r,
                        cnn["b1"].reshape(1, 512), w2r,
                        _pad1(cnn["b2"], 256).reshape(1, 256))

    esm = params["esm"]
    target_supp = _tc_mlp(target_esm2, esm["l1"]["W"],
                          esm["l1"]["b"].reshape(1, 1024),
                          _pad2(esm["l2"]["W"], 1024, 128),
                          _pad1(esm["l2"]["b"], 128).reshape(1, 128))

    d_out = _gat_stack(drug_x, drug_edge_index, drug_batch, drug_supp,
                       params["drug_gat"],
                       [(128, 128), (128, 256), (512, 384), (384, 384)],
                       params["drug_out"])
    t_out = _gat_stack(target_x, target_edge_index, target_batch, target_supp,
                       params["target_gat"],
                       [(128, 128), (128, 128), (256, 256), (256, 256)],
                       params["target_out"])
    return d_out, t_out


# speedup vs baseline: 1.9701x; 1.9701x over previous
import functools

import jax
import jax.numpy as jnp
from jax import lax
from jax.experimental import pallas as pl
from jax.experimental.pallas import tpu as pltpu
from jax.experimental.pallas import tpu_sc as plsc

N_NODES = 10000
NACC = 10240
SPLIT = 5120
HALF = 5376
DUMP = 5120
K = 128
NCH_MAX = 84
ECAP = NCH_MAX * 16 * K
G = 256
CB = 128


def _pad2(a, r, c):
    return jnp.pad(a, ((0, r - a.shape[0]), (0, c - a.shape[1])))


def _pad1(a, n):
    return jnp.pad(a, (0, n - a.shape[0]))


def _sc_gat_body(ncb, h_hbm, asrc_hbm, adst_hbm, s_hbm, d_hbm,
                 out_hbm, ab_v, s_all, dloc_all, dglob_all, e_all,
                 w_v, den_g, rows2_v, wbuf_v, zden_v, den_sp, out_sp,
                 sg0, sg1, ss, sd):
    c = lax.axis_index("c")
    t = lax.axis_index("s")
    sg = (sg0, sg1)
    csplit = c * SPLIT

    pltpu.sync_copy(s_hbm.at[t], s_all)
    pltpu.sync_copy(d_hbm.at[t], dglob_all)

    def _zrow(r, _):
        for u in range(CB // 16):
            wbuf_v[r, pl.ds(u * 16, 16)] = jnp.zeros((16,), jnp.float32)
        return 0
    lax.fori_loop(0, 48, _zrow, 0)
    def _zden(i, _):
        zden_v[pl.ds(i * 16, 16)] = jnp.zeros((16,), jnp.float32)
        return 0
    lax.fori_loop(0, 21, _zden, 0)
    pltpu.sync_copy(zden_v, den_sp.at[pl.ds(t * 336, 336)])
    plsc.subcore_barrier()

    def _fire_g1(j, slot):
        pltpu.async_copy(asrc_hbm.at[s_all.at[j]], ab_v.at[slot, 0],
                         sg[slot])
        pltpu.async_copy(adst_hbm.at[dglob_all.at[j]],
                         ab_v.at[slot, 1], sg[slot])

    _fire_g1(0, 0)

    def _p1(jj, _):
        for hh in range(2):
            j = jj * 2 + hh
            if hh == 0:
                _fire_g1(j + 1, 1)
            else:
                @pl.when(jj < NCH_MAX // 2 - 1)
                def _():
                    _fire_g1(j + 1, 0)
            pltpu.make_async_copy(asrc_hbm.at[s_all.at[j]],
                                  ab_v.at[hh, 0], sg[hh]).wait()
            pltpu.make_async_copy(adst_hbm.at[dglob_all.at[j]],
                                  ab_v.at[hh, 1], sg[hh]).wait()
            for u in range(K // 16):
                sl = pl.ds(u * 16, 16)
                dg = dglob_all[j, sl]
                inb = (dg >= csplit) & (dg < csplit + SPLIT)
                dloc_all[j, sl] = jnp.where(
                    inb, dg - csplit, DUMP + (dg & 255))
                a = ab_v[hh, 0, sl] + ab_v[hh, 1, sl]
                a = jnp.where(a >= 0.0, a, a * 0.2)
                e_all[j, sl] = jnp.exp(a)
            if hh == 0:
                @pl.when(jj > 0)
                def _():
                    pltpu.make_async_copy(
                        e_all.at[j], den_sp.at[dloc_all.at[j]], ss).wait()
            else:
                pltpu.make_async_copy(
                    e_all.at[j], den_sp.at[dloc_all.at[j]], ss).wait()
            pltpu.async_copy(e_all.at[j], den_sp.at[dloc_all.at[j]], ss,
                             add=True)
        return 0
    lax.fori_loop(0, NCH_MAX // 2, _p1, 0)
    pltpu.make_async_copy(e_all.at[0], den_sp.at[dloc_all.at[0]], ss).wait()
    plsc.subcore_barrier()

    for cb in range(ncb):
        for q in range(7):
            pltpu.sync_copy(wbuf_v, out_sp.at[pl.ds(t * 336 + q * 48, 48)])
        plsc.subcore_barrier()

        def _fire_g2(j, slot, cb=cb):
            pltpu.async_copy(h_hbm.at[cb].at[s_all.at[j]],
                             rows2_v.at[slot], sg[slot])

        _fire_g2(0, 0)

        def _p2(jj, _, cb=cb):
            for hh in range(2):
                j = jj * 2 + hh
                if hh == 0:
                    @pl.when(jj > 0)
                    def _():
                        pltpu.make_async_copy(
                            rows2_v.at[1], out_sp.at[dloc_all.at[j]],
                            ss).wait()
                    _fire_g2(j + 1, 1)
                else:
                    pltpu.make_async_copy(
                        rows2_v.at[0], out_sp.at[dloc_all.at[j]], ss).wait()
                    @pl.when(jj < NCH_MAX // 2 - 1)
                    def _():
                        _fire_g2(j + 1, 0)
                pltpu.make_async_copy(h_hbm.at[cb].at[s_all.at[j]],
                                      rows2_v.at[hh], sg[hh]).wait()
                pltpu.async_copy(den_sp.at[dloc_all.at[j]], den_g,
                                 sd).wait()
                for u in range(K // 16):
                    sl = pl.ds(u * 16, 16)
                    w_v[sl] = e_all[j, sl] / (den_g[sl] + 1e-16)
                def _scale(k2, _):
                    for k in (k2 * 2, k2 * 2 + 1):
                        wv = w_v[pl.ds((k // 16) * 16, 16)]
                        wk = jnp.take_along_axis(
                            wv, jnp.full((16,), k % 16, jnp.int32), axis=0)
                        for u in range(CB // 16):
                            sl = pl.ds(u * 16, 16)
                            rows2_v[hh, k, sl] = rows2_v[hh, k, sl] * wk
                    return 0
                lax.fori_loop(0, K // 2, _scale, 0)
                pltpu.async_copy(rows2_v.at[hh], out_sp.at[dloc_all.at[j]],
                                 ss, add=True)
            return 0
        lax.fori_loop(0, NCH_MAX // 2, _p2, 0)
        pltpu.make_async_copy(rows2_v.at[1], out_sp.at[dloc_all.at[0]],
                              ss).wait()
        plsc.subcore_barrier()

        for q in range(7):
            r0 = t * 336 + q * 48
            pltpu.sync_copy(out_sp.at[pl.ds(r0, 48)],
                            rows2_v.at[0].at[pl.ds(0, 48)])
            pltpu.sync_copy(rows2_v.at[0].at[pl.ds(0, 48)],
                            out_hbm.at[c].at[cb].at[pl.ds(r0, 48)])
        plsc.subcore_barrier()


@functools.partial(jax.jit, static_argnums=(0,))
def _sc_gat(ncb, h, asrc, adst, s2, d2):
    mesh = plsc.VectorSubcoreMesh(core_axis_name="c", subcore_axis_name="s",
                                  num_cores=2, num_subcores=16)
    kfn = pl.kernel(
        functools.partial(_sc_gat_body, ncb),
        out_type=jax.ShapeDtypeStruct((2, ncb, HALF, CB), jnp.float32),
        mesh=mesh,
        scratch_types=[
            pltpu.VMEM((2, 2, K), jnp.float32),
            pltpu.VMEM((NCH_MAX, K), jnp.int32),
            pltpu.VMEM((NCH_MAX, K), jnp.int32),
            pltpu.VMEM((NCH_MAX, K), jnp.int32),
            pltpu.VMEM((NCH_MAX, K), jnp.float32),
            pltpu.VMEM((K,), jnp.float32),
            pltpu.VMEM((K,), jnp.float32),
            pltpu.VMEM((2, K, CB), jnp.float32),
            pltpu.VMEM((48, CB), jnp.float32),
            pltpu.VMEM((336,), jnp.float32),
            pltpu.VMEM_SHARED((HALF,), jnp.float32),
            pltpu.VMEM_SHARED((HALF, CB), jnp.float32),
            pltpu.SemaphoreType.DMA,
            pltpu.SemaphoreType.DMA,
            pltpu.SemaphoreType.DMA,
            pltpu.SemaphoreType.DMA,
        ],
        name=f"sc_gat_{ncb}",
    )
    return kfn(h, asrc, adst, s2, d2)


RB = 1024


def _rows_of(p_refs, b_ref):
    z = jnp.concatenate([p[0, 0] for p in p_refs], axis=1)
    return jnp.maximum(z + b_ref[...], 0.0)


def _tc_layer_body(variant, nin, x_refs, h_ref, av_ref):
    cb = pl.program_id(1)
    if variant == "x":
        xin = x_refs[0][...]
        rest = x_refs[1:]
    elif variant == "p":
        xin = _rows_of(x_refs[:nin], x_refs[nin])
        rest = x_refs[nin + 1:]
    else:
        z = _rows_of(x_refs[:nin], x_refs[nin])
        supp_ref, ids_ref = x_refs[nin + 1], x_refs[nin + 2]
        ids = ids_ref[0]
        oh = (lax.broadcasted_iota(jnp.int32, (G, RB), 0) == ids).astype(
            jnp.float32)
        sy = lax.dot_general(oh, supp_ref[...], (((0,), (0,)), ((), ())),
                             preferred_element_type=jnp.float32)
        xin = jnp.concatenate([z + sy, z - sy], axis=1)
        rest = x_refs[nin + 3:]
    w_ref, a8_ref = rest
    h = jnp.dot(xin, w_ref[...], preferred_element_type=jnp.float32)
    h_ref[0] = h
    @pl.when(cb == 0)
    def _():
        av_ref[...] = jnp.zeros_like(av_ref)
    av_ref[...] += lax.dot_general(a8_ref[...], h, (((1,), (1,)), ((), ())),
                                   preferred_element_type=jnp.float32)


def _full(shape):
    return pl.BlockSpec(shape, lambda r, cb: tuple(0 for _ in shape))


def _p_specs(nin):
    return [pl.BlockSpec((1, 1, RB, CB),
                         lambda r, cb, i=i: (r // 5, i, r % 5, 0))
            for i in range(nin)]


def _tc_layer(variant, fin, fout, inputs):
    ncb = fout // CB
    if variant == "x":
        nin = 0
        in_specs = [pl.BlockSpec((RB, fin), lambda r, cb: (r, 0))]
    elif variant == "p":
        nin = fin // CB
        in_specs = _p_specs(nin) + [_full((1, fin))]
    else:
        fs = fin // 2
        nin = fs // CB
        in_specs = _p_specs(nin) + [_full((1, fs)), _full((G, fs)),
                                    pl.BlockSpec((1, 1, RB),
                                                 lambda r, cb: (r, 0, 0))]
    in_specs += [pl.BlockSpec((fin, CB), lambda r, cb: (0, cb)),
                 pl.BlockSpec((8, CB), lambda r, cb: (0, cb))]
    return pl.pallas_call(
        lambda *refs: _tc_layer_body(variant, nin, refs[:-2], refs[-2],
                                     refs[-1]),
        grid=(NACC // RB, ncb),
        in_specs=in_specs,
        out_specs=[pl.BlockSpec((1, RB, CB), lambda r, cb: (cb, r, 0)),
                   pl.BlockSpec((8, RB), lambda r, cb: (0, r))],
        out_shape=[jax.ShapeDtypeStruct((ncb, NACC, CB), jnp.float32),
                   jax.ShapeDtypeStruct((8, NACC), jnp.float32)],
    )(*inputs)


def _pool_body(nin, refs):
    p_refs = refs[:nin]
    b_ref, ids_ref, pooled_ref, cnt_ref = refs[nin:]
    r = pl.program_id(0)
    z = _rows_of(p_refs, b_ref)
    ids = ids_ref[0]
    oh = (lax.broadcasted_iota(jnp.int32, (G, RB), 0) == ids).astype(
        jnp.float32)
    ps = lax.dot_general(oh, z, (((1,), (0,)), ((), ())),
                         preferred_element_type=jnp.float32)
    cs = jnp.sum(oh, axis=1, keepdims=True) * jnp.ones((1, 8), jnp.float32)
    @pl.when(r == 0)
    def _():
        pooled_ref[...] = jnp.zeros_like(pooled_ref)
        cnt_ref[...] = jnp.zeros_like(cnt_ref)
    pooled_ref[...] += ps
    cnt_ref[...] += cs


def _pool_p_specs(nin):
    return [pl.BlockSpec((1, 1, RB, CB),
                         lambda r, i=i: (r // 5, i, r % 5, 0))
            for i in range(nin)]


def _tc_pool(fp, p, b, ids3):
    nin = fp // CB
    return pl.pallas_call(
        lambda *refs: _pool_body(nin, refs),
        grid=(NACC // RB,),
        in_specs=_pool_p_specs(nin) + [
            pl.BlockSpec((1, fp), lambda r: (0, 0)),
            pl.BlockSpec((1, 1, RB), lambda r: (r, 0, 0))],
        out_specs=[pl.BlockSpec((G, fp), lambda r: (0, 0)),
                   pl.BlockSpec((G, 8), lambda r: (0, 0))],
        out_shape=[jax.ShapeDtypeStruct((G, fp), jnp.float32),
                   jax.ShapeDtypeStruct((G, 8), jnp.float32)],
    )(*([p] * nin + [b, ids3]))


def _mlp_body(x_ref, w1_ref, b1_ref, w2_ref, b2_ref, o_ref):
    h = jnp.maximum(jnp.dot(x_ref[...], w1_ref[...],
                            preferred_element_type=jnp.float32)
                    + b1_ref[...], 0.0)
    o_ref[...] = jnp.dot(h, w2_ref[...],
                         preferred_element_type=jnp.float32) + b2_ref[...]


def _tc_mlp(x, w1, b1, w2, b2):
    return pl.pallas_call(
        _mlp_body,
        out_shape=jax.ShapeDtypeStruct((x.shape[0], w2.shape[1]),
                                       jnp.float32),
    )(x, w1, b1, w2, b2)


def _mean_mlp_body(x_ref, cnt_ref, w1_ref, b1_ref, w2_ref, b2_ref, o_ref):
    cnt = jnp.maximum(cnt_ref[:, 0:1], 1.0)
    x = x_ref[...] / cnt
    h = jnp.maximum(jnp.dot(x, w1_ref[...],
                            preferred_element_type=jnp.float32)
                    + b1_ref[...], 0.0)
    o_ref[...] = jnp.dot(h, w2_ref[...],
                         preferred_element_type=jnp.float32) + b2_ref[...]


def _tc_mean_mlp(pooled, cnt, w1, b1, w2, b2):
    return pl.pallas_call(
        _mean_mlp_body,
        out_shape=jax.ShapeDtypeStruct((G, w2.shape[1]), jnp.float32),
    )(pooled, cnt, w1, b1, w2, b2)


def _cnn_body(ids_ref, emb_ref, w1_ref, b1_ref, w2_ref, b2_ref, o_ref):
    ids = ids_ref[0]
    oh = (lax.broadcasted_iota(jnp.int32, (65, 128), 0) == ids).astype(
        jnp.float32)
    xe = lax.dot_general(oh, emb_ref[...], (((0,), (0,)), ((), ())),
                         preferred_element_type=jnp.float32)
    z1 = jnp.zeros((1, 64), jnp.float32)
    xp = jnp.concatenate([z1, xe[0:100], z1], axis=0)
    y1 = jnp.dot(xp[0:99], w1_ref[0], preferred_element_type=jnp.float32)
    for k2 in range(1, 4):
        y1 += jnp.dot(xp[k2:k2 + 99], w1_ref[k2],
                      preferred_element_type=jnp.float32)
    y1 = jnp.maximum(y1 + b1_ref[...], 0.0)
    z2 = jnp.zeros((1, 512), jnp.float32)
    y1p = jnp.concatenate([z2, y1, z2], axis=0)
    y2 = jnp.dot(y1p[0:98], w2_ref[0], preferred_element_type=jnp.float32)
    for k2 in range(1, 4):
        y2 += jnp.dot(y1p[k2:k2 + 98], w2_ref[k2],
                      preferred_element_type=jnp.float32)
    y2 = jnp.maximum(y2 + b2_ref[...], 0.0)
    o_ref[...] = jnp.max(y2[0:97], axis=0, keepdims=True)[None]


def _tc_cnn(ids3, emb, w1r, b1, w2r, b2):
    return pl.pallas_call(
        _cnn_body,
        grid=(G,),
        in_specs=[pl.BlockSpec((1, 1, 128), lambda g: (g, 0, 0)),
                  pl.BlockSpec((65, 64), lambda g: (0, 0)),
                  pl.BlockSpec((4, 64, 512), lambda g: (0, 0, 0)),
                  pl.BlockSpec((1, 512), lambda g: (0, 0)),
                  pl.BlockSpec((4, 512, 256), lambda g: (0, 0, 0)),
                  pl.BlockSpec((1, 256), lambda g: (0, 0))],
        out_specs=pl.BlockSpec((1, 1, 256), lambda g: (g, 0, 0)),
        out_shape=jax.ShapeDtypeStruct((G, 1, 256), jnp.float32),
    )(ids3, emb, w1r, b1, w2r, b2).reshape(G, 256)


def _edge_setup(edge_index):
    src = edge_index[0].astype(jnp.int32)
    dst = edge_index[1].astype(jnp.int32)
    loop = jnp.arange(N_NODES, dtype=jnp.int32)
    npad = NCH_MAX * 16 * K - 2 * N_NODES - src.shape[0] + N_NODES
    ar = jnp.arange(npad, dtype=jnp.int32)
    s_buf = jnp.concatenate([src, loop, 10016 + (ar % 224)])
    d_buf = jnp.concatenate([dst, loop, N_NODES + (ar % 240)])

    def _lay(b):
        return b.reshape(NCH_MAX, 16, K).transpose(1, 0, 2)
    return _lay(s_buf), _lay(d_buf)


def _arrange_w2(w, fs, fsp, fop):
    out = jnp.zeros((2 * fsp, fop), jnp.float32)
    out = out.at[0:fs, 0:w.shape[1]].set(w[0:fs])
    out = out.at[fsp:fsp + fs, 0:w.shape[1]].set(w[fs:2 * fs])
    return out


def _a8(p, fop):
    a = jnp.zeros((8, fop), jnp.float32)
    a = a.at[0, 0:p["a_src"].shape[0]].set(p["a_src"])
    a = a.at[1, 0:p["a_dst"].shape[0]].set(p["a_dst"])
    return a


def _gat_stack(x, edge_index, batch, supp, layers, dims, out_mlp):
    s2, d2 = _edge_setup(edge_index)
    bat3 = jnp.concatenate(
        [batch.astype(jnp.int32),
         jnp.full((NACC - N_NODES,), G, jnp.int32)]).reshape(NACC // RB, 1, RB)

    fin0 = dims[0][0]
    x_p = _pad2(x, NACC, fin0)
    p_half = None
    for i, (fin, fout) in enumerate(dims):
        pp = layers[i]
        w = _pad2(pp["W"], fin, fout)
        if i == 2:
            fs = layers[1]["W"].shape[1]
            w = _arrange_w2(pp["W"], fs, fin // 2, fout)
        a8 = _a8(pp, fout)
        if i == 0:
            h, av = _tc_layer("x", fin, fout, [x_p, w, a8])
        elif i == 2:
            bprev = _pad1(layers[1]["b"], fin // 2).reshape(1, fin // 2)
            h, av = _tc_layer("ps", fin, fout,
                              [p_half] * (fin // 2 // CB)
                              + [bprev, supp, bat3, w, a8])
        else:
            bprev = _pad1(layers[i - 1]["b"], fin).reshape(1, fin)
            h, av = _tc_layer("p", fin, fout,
                              [p_half] * (fin // CB) + [bprev, w, a8])
        p_half = _sc_gat(fout // CB, h, av[0], av[1], s2, d2)

    fp = dims[-1][1]
    b_last = _pad1(layers[-1]["b"], fp).reshape(1, fp)
    pooled, cnt = _tc_pool(fp, p_half, b_last, bat3)
    w1 = _pad2(out_mlp["l1"]["W"], fp, 1024)
    return _tc_mean_mlp(pooled, cnt, w1,
                        out_mlp["l1"]["b"].reshape(1, 1024),
                        out_mlp["l2"]["W"],
                        out_mlp["l2"]["b"].reshape(1, 128))


def kernel(drug_x, drug_edge_index, drug_batch, drug_smiles, target_x,
           target_edge_index, target_batch, target_esm2, params):
    cnn = params["cnn"]
    ids3 = jnp.pad(drug_smiles.astype(jnp.int32),
                   ((0, 0), (0, 28))).reshape(G, 1, 128)
    w1r = jnp.transpose(cnn["w1"], (2, 1, 0))
    w2r = jnp.pad(jnp.transpose(cnn["w2"], (2, 1, 0)),
                  ((0, 0), (0, 0), (0, 100)))
    drug_supp = _tc_cnn(ids3, cnn["emb"], w1r,
                        cnn["b1"].reshape(1, 512), w2r,
                        _pad1(cnn["b2"], 256).reshape(1, 256))

    esm = params["esm"]
    target_supp = _tc_mlp(target_esm2, esm["l1"]["W"],
                          esm["l1"]["b"].reshape(1, 1024),
                          _pad2(esm["l2"]["W"], 1024, 128),
                          _pad1(esm["l2"]["b"], 128).reshape(1, 128))

    d_out = _gat_stack(drug_x, drug_edge_index, drug_batch, drug_supp,
                       params["drug_gat"],
                       [(128, 128), (128, 256), (512, 384), (384, 384)],
                       params["drug_out"])
    t_out = _gat_stack(target_x, target_edge_index, target_batch, target_supp,
                       params["target_gat"],
                       [(128, 128), (128, 128), (256, 256), (256, 256)],
                       params["target_out"])
    return d_out, t_out


# speedup vs baseline: 2.0131x; 1.0218x over previous
import functools

import jax
import jax.numpy as jnp
from jax import lax
from jax.experimental import pallas as pl
from jax.experimental.pallas import tpu as pltpu
from jax.experimental.pallas import tpu_sc as plsc

N_NODES = 10000
NACC = 10240
SPLIT = 5120
HALF = 5376
DUMP = 5120
K = 128
NCH_MAX = 84
ECAP = NCH_MAX * 16 * K
G = 256
CB = 128


def _pad2(a, r, c):
    return jnp.pad(a, ((0, r - a.shape[0]), (0, c - a.shape[1])))


def _pad1(a, n):
    return jnp.pad(a, (0, n - a.shape[0]))


def _sc_gat_body(ncb, h_hbm, asrc_hbm, adst_hbm, s_hbm, d_hbm,
                 out_hbm, ab_v, s_all, dloc_all, dglob_all, e_all,
                 w_v, den_g, rows2_v, wbuf_v, zden_v, den_sp, out_sp,
                 sg0, sg1, ss, sd):
    c = lax.axis_index("c")
    t = lax.axis_index("s")
    sg = (sg0, sg1)
    csplit = c * SPLIT

    pltpu.sync_copy(s_hbm.at[t], s_all)
    pltpu.sync_copy(d_hbm.at[t], dglob_all)

    def _zrow(r, _):
        for u in range(CB // 16):
            wbuf_v[r, pl.ds(u * 16, 16)] = jnp.zeros((16,), jnp.float32)
        return 0
    lax.fori_loop(0, 48, _zrow, 0)
    def _zden(i, _):
        zden_v[pl.ds(i * 16, 16)] = jnp.zeros((16,), jnp.float32)
        return 0
    lax.fori_loop(0, 21, _zden, 0)
    pltpu.sync_copy(zden_v, den_sp.at[pl.ds(t * 336, 336)])
    plsc.subcore_barrier()

    def _fire_g1(j, slot):
        pltpu.async_copy(asrc_hbm.at[s_all.at[j]], ab_v.at[slot, 0],
                         sg[slot])
        pltpu.async_copy(adst_hbm.at[dglob_all.at[j]],
                         ab_v.at[slot, 1], sg[slot])

    _fire_g1(0, 0)

    def _p1(jj, _):
        for hh in range(2):
            j = jj * 2 + hh
            if hh == 0:
                _fire_g1(j + 1, 1)
            else:
                @pl.when(jj < NCH_MAX // 2 - 1)
                def _():
                    _fire_g1(j + 1, 0)
            pltpu.make_async_copy(asrc_hbm.at[s_all.at[j]],
                                  ab_v.at[hh, 0], sg[hh]).wait()
            pltpu.make_async_copy(adst_hbm.at[dglob_all.at[j]],
                                  ab_v.at[hh, 1], sg[hh]).wait()
            for u in range(K // 16):
                sl = pl.ds(u * 16, 16)
                dg = dglob_all[j, sl]
                inb = (dg >= csplit) & (dg < csplit + SPLIT)
                dloc_all[j, sl] = jnp.where(
                    inb, dg - csplit, DUMP + (dg & 255))
                a = ab_v[hh, 0, sl] + ab_v[hh, 1, sl]
                a = jnp.where(a >= 0.0, a, a * 0.2)
                e_all[j, sl] = jnp.exp(a)
            if hh == 0:
                @pl.when(jj > 0)
                def _():
                    pltpu.make_async_copy(
                        e_all.at[j], den_sp.at[dloc_all.at[j]], ss).wait()
            else:
                pltpu.make_async_copy(
                    e_all.at[j], den_sp.at[dloc_all.at[j]], ss).wait()
            pltpu.async_copy(e_all.at[j], den_sp.at[dloc_all.at[j]], ss,
                             add=True)
        return 0
    lax.fori_loop(0, NCH_MAX // 2, _p1, 0)
    pltpu.make_async_copy(e_all.at[0], den_sp.at[dloc_all.at[0]], ss).wait()
    plsc.subcore_barrier()

    def _pw(j, _):
        pltpu.async_copy(den_sp.at[dloc_all.at[j]], den_g, sd).wait()
        for u in range(K // 16):
            sl = pl.ds(u * 16, 16)
            e_all[j, sl] = e_all[j, sl] / (den_g[sl] + 1e-16)
        return 0
    lax.fori_loop(0, NCH_MAX, _pw, 0)

    for cb in range(ncb):
        for q in range(7):
            pltpu.sync_copy(wbuf_v, out_sp.at[pl.ds(t * 336 + q * 48, 48)])
        plsc.subcore_barrier()

        def _fire_g2(j, slot, cb=cb):
            pltpu.async_copy(h_hbm.at[cb].at[s_all.at[j]],
                             rows2_v.at[slot], sg[slot])

        _fire_g2(0, 0)

        def _p2(jj, _, cb=cb):
            for hh in range(2):
                j = jj * 2 + hh
                if hh == 0:
                    @pl.when(jj > 0)
                    def _():
                        pltpu.make_async_copy(
                            rows2_v.at[1], out_sp.at[dloc_all.at[j]],
                            ss).wait()
                    _fire_g2(j + 1, 1)
                else:
                    pltpu.make_async_copy(
                        rows2_v.at[0], out_sp.at[dloc_all.at[j]], ss).wait()
                    @pl.when(jj < NCH_MAX // 2 - 1)
                    def _():
                        _fire_g2(j + 1, 0)
                pltpu.make_async_copy(h_hbm.at[cb].at[s_all.at[j]],
                                      rows2_v.at[hh], sg[hh]).wait()
                for u in range(K // 16):
                    sl = pl.ds(u * 16, 16)
                    w_v[sl] = e_all[j, sl]
                def _scale(k2, _):
                    for k in (k2 * 2, k2 * 2 + 1):
                        wv = w_v[pl.ds((k // 16) * 16, 16)]
                        wk = jnp.take_along_axis(
                            wv, jnp.full((16,), k % 16, jnp.int32), axis=0)
                        for u in range(CB // 16):
                            sl = pl.ds(u * 16, 16)
                            rows2_v[hh, k, sl] = rows2_v[hh, k, sl] * wk
                    return 0
                lax.fori_loop(0, K // 2, _scale, 0)
                pltpu.async_copy(rows2_v.at[hh], out_sp.at[dloc_all.at[j]],
                                 ss, add=True)
            return 0
        lax.fori_loop(0, NCH_MAX // 2, _p2, 0)
        pltpu.make_async_copy(rows2_v.at[1], out_sp.at[dloc_all.at[0]],
                              ss).wait()
        plsc.subcore_barrier()

        for q in range(7):
            r0 = t * 336 + q * 48
            pltpu.sync_copy(out_sp.at[pl.ds(r0, 48)],
                            rows2_v.at[0].at[pl.ds(0, 48)])
            pltpu.sync_copy(rows2_v.at[0].at[pl.ds(0, 48)],
                            out_hbm.at[c].at[cb].at[pl.ds(r0, 48)])
        plsc.subcore_barrier()


@functools.partial(jax.jit, static_argnums=(0,))
def _sc_gat(ncb, h, asrc, adst, s2, d2):
    mesh = plsc.VectorSubcoreMesh(core_axis_name="c", subcore_axis_name="s",
                                  num_cores=2, num_subcores=16)
    kfn = pl.kernel(
        functools.partial(_sc_gat_body, ncb),
        out_type=jax.ShapeDtypeStruct((2, ncb, HALF, CB), jnp.float32),
        mesh=mesh,
        scratch_types=[
            pltpu.VMEM((2, 2, K), jnp.float32),
            pltpu.VMEM((NCH_MAX, K), jnp.int32),
            pltpu.VMEM((NCH_MAX, K), jnp.int32),
            pltpu.VMEM((NCH_MAX, K), jnp.int32),
            pltpu.VMEM((NCH_MAX, K), jnp.float32),
            pltpu.VMEM((K,), jnp.float32),
            pltpu.VMEM((K,), jnp.float32),
            pltpu.VMEM((2, K, CB), jnp.float32),
            pltpu.VMEM((48, CB), jnp.float32),
            pltpu.VMEM((336,), jnp.float32),
            pltpu.VMEM_SHARED((HALF,), jnp.float32),
            pltpu.VMEM_SHARED((HALF, CB), jnp.float32),
            pltpu.SemaphoreType.DMA,
            pltpu.SemaphoreType.DMA,
            pltpu.SemaphoreType.DMA,
            pltpu.SemaphoreType.DMA,
        ],
        name=f"sc_gat_{ncb}",
    )
    return kfn(h, asrc, adst, s2, d2)


RB = 1024


def _rows_of(p_refs, b_ref):
    z = jnp.concatenate([p[0, 0] for p in p_refs], axis=1)
    return jnp.maximum(z + b_ref[...], 0.0)


def _tc_layer_body(variant, nin, x_refs, h_ref, av_ref):
    cb = pl.program_id(1)
    if variant == "x":
        xin = x_refs[0][...]
        rest = x_refs[1:]
    elif variant == "p":
        xin = _rows_of(x_refs[:nin], x_refs[nin])
        rest = x_refs[nin + 1:]
    else:
        z = _rows_of(x_refs[:nin], x_refs[nin])
        supp_ref, ids_ref = x_refs[nin + 1], x_refs[nin + 2]
        ids = ids_ref[0]
        oh = (lax.broadcasted_iota(jnp.int32, (G, RB), 0) == ids).astype(
            jnp.float32)
        sy = lax.dot_general(oh, supp_ref[...], (((0,), (0,)), ((), ())),
                             preferred_element_type=jnp.float32)
        xin = jnp.concatenate([z + sy, z - sy], axis=1)
        rest = x_refs[nin + 3:]
    w_ref, a8_ref = rest
    h = jnp.dot(xin, w_ref[...], preferred_element_type=jnp.float32)
    h_ref[0] = h
    @pl.when(cb == 0)
    def _():
        av_ref[...] = jnp.zeros_like(av_ref)
    av_ref[...] += lax.dot_general(a8_ref[...], h, (((1,), (1,)), ((), ())),
                                   preferred_element_type=jnp.float32)


def _full(shape):
    return pl.BlockSpec(shape, lambda r, cb: tuple(0 for _ in shape))


def _p_specs(nin):
    return [pl.BlockSpec((1, 1, RB, CB),
                         lambda r, cb, i=i: (r // 5, i, r % 5, 0))
            for i in range(nin)]


def _tc_layer(variant, fin, fout, inputs):
    ncb = fout // CB
    if variant == "x":
        nin = 0
        in_specs = [pl.BlockSpec((RB, fin), lambda r, cb: (r, 0))]
    elif variant == "p":
        nin = fin // CB
        in_specs = _p_specs(nin) + [_full((1, fin))]
    else:
        fs = fin // 2
        nin = fs // CB
        in_specs = _p_specs(nin) + [_full((1, fs)), _full((G, fs)),
                                    pl.BlockSpec((1, 1, RB),
                                                 lambda r, cb: (r, 0, 0))]
    in_specs += [pl.BlockSpec((fin, CB), lambda r, cb: (0, cb)),
                 pl.BlockSpec((8, CB), lambda r, cb: (0, cb))]
    return pl.pallas_call(
        lambda *refs: _tc_layer_body(variant, nin, refs[:-2], refs[-2],
                                     refs[-1]),
        grid=(NACC // RB, ncb),
        in_specs=in_specs,
        out_specs=[pl.BlockSpec((1, RB, CB), lambda r, cb: (cb, r, 0)),
                   pl.BlockSpec((8, RB), lambda r, cb: (0, r))],
        out_shape=[jax.ShapeDtypeStruct((ncb, NACC, CB), jnp.float32),
                   jax.ShapeDtypeStruct((8, NACC), jnp.float32)],
    )(*inputs)


def _pool_body(nin, refs):
    p_refs = refs[:nin]
    b_ref, ids_ref, pooled_ref, cnt_ref = refs[nin:]
    r = pl.program_id(0)
    z = _rows_of(p_refs, b_ref)
    ids = ids_ref[0]
    oh = (lax.broadcasted_iota(jnp.int32, (G, RB), 0) == ids).astype(
        jnp.float32)
    ps = lax.dot_general(oh, z, (((1,), (0,)), ((), ())),
                         preferred_element_type=jnp.float32)
    cs = jnp.sum(oh, axis=1, keepdims=True) * jnp.ones((1, 8), jnp.float32)
    @pl.when(r == 0)
    def _():
        pooled_ref[...] = jnp.zeros_like(pooled_ref)
        cnt_ref[...] = jnp.zeros_like(cnt_ref)
    pooled_ref[...] += ps
    cnt_ref[...] += cs


def _pool_p_specs(nin):
    return [pl.BlockSpec((1, 1, RB, CB),
                         lambda r, i=i: (r // 5, i, r % 5, 0))
            for i in range(nin)]


def _tc_pool(fp, p, b, ids3):
    nin = fp // CB
    return pl.pallas_call(
        lambda *refs: _pool_body(nin, refs),
        grid=(NACC // RB,),
        in_specs=_pool_p_specs(nin) + [
            pl.BlockSpec((1, fp), lambda r: (0, 0)),
            pl.BlockSpec((1, 1, RB), lambda r: (r, 0, 0))],
        out_specs=[pl.BlockSpec((G, fp), lambda r: (0, 0)),
                   pl.BlockSpec((G, 8), lambda r: (0, 0))],
        out_shape=[jax.ShapeDtypeStruct((G, fp), jnp.float32),
                   jax.ShapeDtypeStruct((G, 8), jnp.float32)],
    )(*([p] * nin + [b, ids3]))


def _mlp_body(x_ref, w1_ref, b1_ref, w2_ref, b2_ref, o_ref):
    h = jnp.maximum(jnp.dot(x_ref[...], w1_ref[...],
                            preferred_element_type=jnp.float32)
                    + b1_ref[...], 0.0)
    o_ref[...] = jnp.dot(h, w2_ref[...],
                         preferred_element_type=jnp.float32) + b2_ref[...]


def _tc_mlp(x, w1, b1, w2, b2):
    return pl.pallas_call(
        _mlp_body,
        out_shape=jax.ShapeDtypeStruct((x.shape[0], w2.shape[1]),
                                       jnp.float32),
    )(x, w1, b1, w2, b2)


def _mean_mlp_body(x_ref, cnt_ref, w1_ref, b1_ref, w2_ref, b2_ref, o_ref):
    cnt = jnp.maximum(cnt_ref[:, 0:1], 1.0)
    x = x_ref[...] / cnt
    h = jnp.maximum(jnp.dot(x, w1_ref[...],
                            preferred_element_type=jnp.float32)
                    + b1_ref[...], 0.0)
    o_ref[...] = jnp.dot(h, w2_ref[...],
                         preferred_element_type=jnp.float32) + b2_ref[...]


def _tc_mean_mlp(pooled, cnt, w1, b1, w2, b2):
    return pl.pallas_call(
        _mean_mlp_body,
        out_shape=jax.ShapeDtypeStruct((G, w2.shape[1]), jnp.float32),
    )(pooled, cnt, w1, b1, w2, b2)


def _cnn_body(ids_ref, emb_ref, w1_ref, b1_ref, w2_ref, b2_ref, o_ref):
    ids = ids_ref[0]
    oh = (lax.broadcasted_iota(jnp.int32, (65, 128), 0) == ids).astype(
        jnp.float32)
    xe = lax.dot_general(oh, emb_ref[...], (((0,), (0,)), ((), ())),
                         preferred_element_type=jnp.float32)
    z1 = jnp.zeros((1, 64), jnp.float32)
    xp = jnp.concatenate([z1, xe[0:100], z1], axis=0)
    y1 = jnp.dot(xp[0:99], w1_ref[0], preferred_element_type=jnp.float32)
    for k2 in range(1, 4):
        y1 += jnp.dot(xp[k2:k2 + 99], w1_ref[k2],
                      preferred_element_type=jnp.float32)
    y1 = jnp.maximum(y1 + b1_ref[...], 0.0)
    z2 = jnp.zeros((1, 512), jnp.float32)
    y1p = jnp.concatenate([z2, y1, z2], axis=0)
    y2 = jnp.dot(y1p[0:98], w2_ref[0], preferred_element_type=jnp.float32)
    for k2 in range(1, 4):
        y2 += jnp.dot(y1p[k2:k2 + 98], w2_ref[k2],
                      preferred_element_type=jnp.float32)
    y2 = jnp.maximum(y2 + b2_ref[...], 0.0)
    o_ref[...] = jnp.max(y2[0:97], axis=0, keepdims=True)[None]


def _tc_cnn(ids3, emb, w1r, b1, w2r, b2):
    return pl.pallas_call(
        _cnn_body,
        grid=(G,),
        in_specs=[pl.BlockSpec((1, 1, 128), lambda g: (g, 0, 0)),
                  pl.BlockSpec((65, 64), lambda g: (0, 0)),
                  pl.BlockSpec((4, 64, 512), lambda g: (0, 0, 0)),
                  pl.BlockSpec((1, 512), lambda g: (0, 0)),
                  pl.BlockSpec((4, 512, 256), lambda g: (0, 0, 0)),
                  pl.BlockSpec((1, 256), lambda g: (0, 0))],
        out_specs=pl.BlockSpec((1, 1, 256), lambda g: (g, 0, 0)),
        out_shape=jax.ShapeDtypeStruct((G, 1, 256), jnp.float32),
    )(ids3, emb, w1r, b1, w2r, b2).reshape(G, 256)


def _edge_setup(edge_index):
    src = edge_index[0].astype(jnp.int32)
    dst = edge_index[1].astype(jnp.int32)
    loop = jnp.arange(N_NODES, dtype=jnp.int32)
    npad = NCH_MAX * 16 * K - 2 * N_NODES - src.shape[0] + N_NODES
    ar = jnp.arange(npad, dtype=jnp.int32)
    s_buf = jnp.concatenate([src, loop, 10016 + (ar % 224)])
    d_buf = jnp.concatenate([dst, loop, N_NODES + (ar % 240)])

    def _lay(b):
        return b.reshape(NCH_MAX, 16, K).transpose(1, 0, 2)
    return _lay(s_buf), _lay(d_buf)


def _arrange_w2(w, fs, fsp, fop):
    out = jnp.zeros((2 * fsp, fop), jnp.float32)
    out = out.at[0:fs, 0:w.shape[1]].set(w[0:fs])
    out = out.at[fsp:fsp + fs, 0:w.shape[1]].set(w[fs:2 * fs])
    return out


def _a8(p, fop):
    a = jnp.zeros((8, fop), jnp.float32)
    a = a.at[0, 0:p["a_src"].shape[0]].set(p["a_src"])
    a = a.at[1, 0:p["a_dst"].shape[0]].set(p["a_dst"])
    return a


def _gat_stack(x, edge_index, batch, supp, layers, dims, out_mlp):
    s2, d2 = _edge_setup(edge_index)
    bat3 = jnp.concatenate(
        [batch.astype(jnp.int32),
         jnp.full((NACC - N_NODES,), G, jnp.int32)]).reshape(NACC // RB, 1, RB)

    fin0 = dims[0][0]
    x_p = _pad2(x, NACC, fin0)
    p_half = None
    for i, (fin, fout) in enumerate(dims):
        pp = layers[i]
        w = _pad2(pp["W"], fin, fout)
        if i == 2:
            fs = layers[1]["W"].shape[1]
            w = _arrange_w2(pp["W"], fs, fin // 2, fout)
        a8 = _a8(pp, fout)
        if i == 0:
            h, av = _tc_layer("x", fin, fout, [x_p, w, a8])
        elif i == 2:
            bprev = _pad1(layers[1]["b"], fin // 2).reshape(1, fin // 2)
            h, av = _tc_layer("ps", fin, fout,
                              [p_half] * (fin // 2 // CB)
                              + [bprev, supp, bat3, w, a8])
        else:
            bprev = _pad1(layers[i - 1]["b"], fin).reshape(1, fin)
            h, av = _tc_layer("p", fin, fout,
                              [p_half] * (fin // CB) + [bprev, w, a8])
        p_half = _sc_gat(fout // CB, h, av[0], av[1], s2, d2)

    fp = dims[-1][1]
    b_last = _pad1(layers[-1]["b"], fp).reshape(1, fp)
    pooled, cnt = _tc_pool(fp, p_half, b_last, bat3)
    w1 = _pad2(out_mlp["l1"]["W"], fp, 1024)
    return _tc_mean_mlp(pooled, cnt, w1,
                        out_mlp["l1"]["b"].reshape(1, 1024),
                        out_mlp["l2"]["W"],
                        out_mlp["l2"]["b"].reshape(1, 128))


def kernel(drug_x, drug_edge_index, drug_batch, drug_smiles, target_x,
           target_edge_index, target_batch, target_esm2, params):
    cnn = params["cnn"]
    ids3 = jnp.pad(drug_smiles.astype(jnp.int32),
                   ((0, 0), (0, 28))).reshape(G, 1, 128)
    w1r = jnp.transpose(cnn["w1"], (2, 1, 0))
    w2r = jnp.pad(jnp.transpose(cnn["w2"], (2, 1, 0)),
                  ((0, 0), (0, 0), (0, 100)))
    drug_supp = _tc_cnn(ids3, cnn["emb"], w1r,
                        cnn["b1"].reshape(1, 512), w2r,
                        _pad1(cnn["b2"], 256).reshape(1, 256))

    esm = params["esm"]
    target_supp = _tc_mlp(target_esm2, esm["l1"]["W"],
                          esm["l1"]["b"].reshape(1, 1024),
                          _pad2(esm["l2"]["W"], 1024, 128),
                          _pad1(esm["l2"]["b"], 128).reshape(1, 128))

    d_out = _gat_stack(drug_x, drug_edge_index, drug_batch, drug_supp,
                       params["drug_gat"],
                       [(128, 128), (128, 256), (512, 384), (384, 384)],
                       params["drug_out"])
    t_out = _gat_stack(target_x, target_edge_index, target_batch, target_supp,
                       params["target_gat"],
                       [(128, 128), (128, 128), (256, 256), (256, 256)],
                       params["target_out"])
    return d_out, t_out


# speedup vs baseline: 2.0212x; 1.0040x over previous
import functools

import jax
import jax.numpy as jnp
from jax import lax
from jax.experimental import pallas as pl
from jax.experimental.pallas import tpu as pltpu
from jax.experimental.pallas import tpu_sc as plsc

N_NODES = 10000
NACC = 10240
SPLIT = 5120
HALF = 5376
DUMP = 5120
K = 128
NCH_MAX = 84
ECAP = NCH_MAX * 16 * K
G = 256
CB = 128


def _pad2(a, r, c):
    return jnp.pad(a, ((0, r - a.shape[0]), (0, c - a.shape[1])))


def _pad1(a, n):
    return jnp.pad(a, (0, n - a.shape[0]))


def _sc_gat_body(ncb, h_hbm, asrc_hbm, adst_hbm, s_hbm, d_hbm,
                 out_hbm, ab_v, s_all, dloc_all, dglob_all, e_all,
                 w_v, den_g2, rows2_v, wbuf_v, zden_v, den_sp, out_sp,
                 sg0, sg1, ss, sd):
    c = lax.axis_index("c")
    t = lax.axis_index("s")
    sg = (sg0, sg1)
    csplit = c * SPLIT

    pltpu.sync_copy(s_hbm.at[t], s_all)
    pltpu.sync_copy(d_hbm.at[t], dglob_all)

    def _zrow(r, _):
        for u in range(CB // 16):
            wbuf_v[r, pl.ds(u * 16, 16)] = jnp.zeros((16,), jnp.float32)
        return 0
    lax.fori_loop(0, 48, _zrow, 0)
    def _zden(i, _):
        zden_v[pl.ds(i * 16, 16)] = jnp.zeros((16,), jnp.float32)
        return 0
    lax.fori_loop(0, 21, _zden, 0)
    pltpu.sync_copy(zden_v, den_sp.at[pl.ds(t * 336, 336)])
    plsc.subcore_barrier()

    def _fire_g1(j, slot):
        pltpu.async_copy(asrc_hbm.at[s_all.at[j]], ab_v.at[slot, 0],
                         sg[slot])
        pltpu.async_copy(adst_hbm.at[dglob_all.at[j]],
                         ab_v.at[slot, 1], sg[slot])

    _fire_g1(0, 0)

    def _p1(jj, _):
        for hh in range(2):
            j = jj * 2 + hh
            if hh == 0:
                _fire_g1(j + 1, 1)
            else:
                @pl.when(jj < NCH_MAX // 2 - 1)
                def _():
                    _fire_g1(j + 1, 0)
            pltpu.make_async_copy(asrc_hbm.at[s_all.at[j]],
                                  ab_v.at[hh, 0], sg[hh]).wait()
            pltpu.make_async_copy(adst_hbm.at[dglob_all.at[j]],
                                  ab_v.at[hh, 1], sg[hh]).wait()
            for u in range(K // 16):
                sl = pl.ds(u * 16, 16)
                dg = dglob_all[j, sl]
                inb = (dg >= csplit) & (dg < csplit + SPLIT)
                dloc_all[j, sl] = jnp.where(
                    inb, dg - csplit, DUMP + (dg & 255))
                a = ab_v[hh, 0, sl] + ab_v[hh, 1, sl]
                a = jnp.where(a >= 0.0, a, a * 0.2)
                e_all[j, sl] = jnp.exp(a)
            if hh == 0:
                @pl.when(jj > 0)
                def _():
                    pltpu.make_async_copy(
                        e_all.at[j], den_sp.at[dloc_all.at[j]], ss).wait()
            else:
                pltpu.make_async_copy(
                    e_all.at[j], den_sp.at[dloc_all.at[j]], ss).wait()
            pltpu.async_copy(e_all.at[j], den_sp.at[dloc_all.at[j]], ss,
                             add=True)
        return 0
    lax.fori_loop(0, NCH_MAX // 2, _p1, 0)
    pltpu.make_async_copy(e_all.at[0], den_sp.at[dloc_all.at[0]], ss).wait()
    plsc.subcore_barrier()

    def _fire_d(j, slot):
        pltpu.async_copy(den_sp.at[dloc_all.at[j]], den_g2.at[slot],
                         sg[slot])

    _fire_d(0, 0)

    def _pw(jj, _):
        for hh in range(2):
            j = jj * 2 + hh
            if hh == 0:
                _fire_d(j + 1, 1)
            else:
                @pl.when(jj < NCH_MAX // 2 - 1)
                def _():
                    _fire_d(j + 1, 0)
            pltpu.make_async_copy(den_sp.at[dloc_all.at[j]],
                                  den_g2.at[hh], sg[hh]).wait()
            for u in range(K // 16):
                sl = pl.ds(u * 16, 16)
                e_all[j, sl] = e_all[j, sl] / (den_g2[hh, sl] + 1e-16)
        return 0
    lax.fori_loop(0, NCH_MAX // 2, _pw, 0)

    for cb in range(ncb):
        for q in range(7):
            pltpu.sync_copy(wbuf_v, out_sp.at[pl.ds(t * 336 + q * 48, 48)])
        plsc.subcore_barrier()

        def _fire_g2(j, slot, cb=cb):
            pltpu.async_copy(h_hbm.at[cb].at[s_all.at[j]],
                             rows2_v.at[slot], sg[slot])

        _fire_g2(0, 0)

        def _p2(jj, _, cb=cb):
            for hh in range(2):
                j = jj * 2 + hh
                if hh == 0:
                    @pl.when(jj > 0)
                    def _():
                        pltpu.make_async_copy(
                            rows2_v.at[1], out_sp.at[dloc_all.at[j]],
                            ss).wait()
                    _fire_g2(j + 1, 1)
                else:
                    pltpu.make_async_copy(
                        rows2_v.at[0], out_sp.at[dloc_all.at[j]], ss).wait()
                    @pl.when(jj < NCH_MAX // 2 - 1)
                    def _():
                        _fire_g2(j + 1, 0)
                pltpu.make_async_copy(h_hbm.at[cb].at[s_all.at[j]],
                                      rows2_v.at[hh], sg[hh]).wait()
                for u in range(K // 16):
                    sl = pl.ds(u * 16, 16)
                    w_v[sl] = e_all[j, sl]
                def _scale(k2, _):
                    for k in (k2 * 4, k2 * 4 + 1, k2 * 4 + 2, k2 * 4 + 3):
                        wv = w_v[pl.ds((k // 16) * 16, 16)]
                        wk = jnp.take_along_axis(
                            wv, jnp.full((16,), k % 16, jnp.int32), axis=0)
                        for u in range(CB // 16):
                            sl = pl.ds(u * 16, 16)
                            rows2_v[hh, k, sl] = rows2_v[hh, k, sl] * wk
                    return 0
                lax.fori_loop(0, K // 4, _scale, 0)
                pltpu.async_copy(rows2_v.at[hh], out_sp.at[dloc_all.at[j]],
                                 ss, add=True)
            return 0
        lax.fori_loop(0, NCH_MAX // 2, _p2, 0)
        pltpu.make_async_copy(rows2_v.at[1], out_sp.at[dloc_all.at[0]],
                              ss).wait()
        plsc.subcore_barrier()

        for q in range(7):
            r0 = t * 336 + q * 48
            pltpu.sync_copy(out_sp.at[pl.ds(r0, 48)],
                            rows2_v.at[0].at[pl.ds(0, 48)])
            pltpu.sync_copy(rows2_v.at[0].at[pl.ds(0, 48)],
                            out_hbm.at[c].at[cb].at[pl.ds(r0, 48)])
        plsc.subcore_barrier()


@functools.partial(jax.jit, static_argnums=(0,))
def _sc_gat(ncb, h, asrc, adst, s2, d2):
    mesh = plsc.VectorSubcoreMesh(core_axis_name="c", subcore_axis_name="s",
                                  num_cores=2, num_subcores=16)
    kfn = pl.kernel(
        functools.partial(_sc_gat_body, ncb),
        out_type=jax.ShapeDtypeStruct((2, ncb, HALF, CB), jnp.float32),
        mesh=mesh,
        scratch_types=[
            pltpu.VMEM((2, 2, K), jnp.float32),
            pltpu.VMEM((NCH_MAX, K), jnp.int32),
            pltpu.VMEM((NCH_MAX, K), jnp.int32),
            pltpu.VMEM((NCH_MAX, K), jnp.int32),
            pltpu.VMEM((NCH_MAX, K), jnp.float32),
            pltpu.VMEM((K,), jnp.float32),
            pltpu.VMEM((2, K), jnp.float32),
            pltpu.VMEM((2, K, CB), jnp.float32),
            pltpu.VMEM((48, CB), jnp.float32),
            pltpu.VMEM((336,), jnp.float32),
            pltpu.VMEM_SHARED((HALF,), jnp.float32),
            pltpu.VMEM_SHARED((HALF, CB), jnp.float32),
            pltpu.SemaphoreType.DMA,
            pltpu.SemaphoreType.DMA,
            pltpu.SemaphoreType.DMA,
            pltpu.SemaphoreType.DMA,
        ],
        name=f"sc_gat_{ncb}",
    )
    return kfn(h, asrc, adst, s2, d2)


RB = 1024


def _rows_of(p_refs, b_ref):
    z = jnp.concatenate([p[0, 0] for p in p_refs], axis=1)
    return jnp.maximum(z + b_ref[...], 0.0)


def _tc_layer_body(variant, nin, x_refs, h_ref, av_ref):
    cb = pl.program_id(1)
    if variant == "x":
        xin = x_refs[0][...]
        rest = x_refs[1:]
    elif variant == "p":
        xin = _rows_of(x_refs[:nin], x_refs[nin])
        rest = x_refs[nin + 1:]
    else:
        z = _rows_of(x_refs[:nin], x_refs[nin])
        supp_ref, ids_ref = x_refs[nin + 1], x_refs[nin + 2]
        ids = ids_ref[0]
        oh = (lax.broadcasted_iota(jnp.int32, (G, RB), 0) == ids).astype(
            jnp.float32)
        sy = lax.dot_general(oh, supp_ref[...], (((0,), (0,)), ((), ())),
                             preferred_element_type=jnp.float32)
        xin = jnp.concatenate([z + sy, z - sy], axis=1)
        rest = x_refs[nin + 3:]
    w_ref, a8_ref = rest
    h = jnp.dot(xin, w_ref[...], preferred_element_type=jnp.float32)
    h_ref[0] = h
    @pl.when(cb == 0)
    def _():
        av_ref[...] = jnp.zeros_like(av_ref)
    av_ref[...] += lax.dot_general(a8_ref[...], h, (((1,), (1,)), ((), ())),
                                   preferred_element_type=jnp.float32)


def _full(shape):
    return pl.BlockSpec(shape, lambda r, cb: tuple(0 for _ in shape))


def _p_specs(nin):
    return [pl.BlockSpec((1, 1, RB, CB),
                         lambda r, cb, i=i: (r // 5, i, r % 5, 0))
            for i in range(nin)]


def _tc_layer(variant, fin, fout, inputs):
    ncb = fout // CB
    if variant == "x":
        nin = 0
        in_specs = [pl.BlockSpec((RB, fin), lambda r, cb: (r, 0))]
    elif variant == "p":
        nin = fin // CB
        in_specs = _p_specs(nin) + [_full((1, fin))]
    else:
        fs = fin // 2
        nin = fs // CB
        in_specs = _p_specs(nin) + [_full((1, fs)), _full((G, fs)),
                                    pl.BlockSpec((1, 1, RB),
                                                 lambda r, cb: (r, 0, 0))]
    in_specs += [pl.BlockSpec((fin, CB), lambda r, cb: (0, cb)),
                 pl.BlockSpec((8, CB), lambda r, cb: (0, cb))]
    return pl.pallas_call(
        lambda *refs: _tc_layer_body(variant, nin, refs[:-2], refs[-2],
                                     refs[-1]),
        grid=(NACC // RB, ncb),
        in_specs=in_specs,
        out_specs=[pl.BlockSpec((1, RB, CB), lambda r, cb: (cb, r, 0)),
                   pl.BlockSpec((8, RB), lambda r, cb: (0, r))],
        out_shape=[jax.ShapeDtypeStruct((ncb, NACC, CB), jnp.float32),
                   jax.ShapeDtypeStruct((8, NACC), jnp.float32)],
    )(*inputs)


def _pool_body(nin, refs):
    p_refs = refs[:nin]
    b_ref, ids_ref, pooled_ref, cnt_ref = refs[nin:]
    r = pl.program_id(0)
    z = _rows_of(p_refs, b_ref)
    ids = ids_ref[0]
    oh = (lax.broadcasted_iota(jnp.int32, (G, RB), 0) == ids).astype(
        jnp.float32)
    ps = lax.dot_general(oh, z, (((1,), (0,)), ((), ())),
                         preferred_element_type=jnp.float32)
    cs = jnp.sum(oh, axis=1, keepdims=True) * jnp.ones((1, 8), jnp.float32)
    @pl.when(r == 0)
    def _():
        pooled_ref[...] = jnp.zeros_like(pooled_ref)
        cnt_ref[...] = jnp.zeros_like(cnt_ref)
    pooled_ref[...] += ps
    cnt_ref[...] += cs


def _pool_p_specs(nin):
    return [pl.BlockSpec((1, 1, RB, CB),
                         lambda r, i=i: (r // 5, i, r % 5, 0))
            for i in range(nin)]


def _tc_pool(fp, p, b, ids3):
    nin = fp // CB
    return pl.pallas_call(
        lambda *refs: _pool_body(nin, refs),
        grid=(NACC // RB,),
        in_specs=_pool_p_specs(nin) + [
            pl.BlockSpec((1, fp), lambda r: (0, 0)),
            pl.BlockSpec((1, 1, RB), lambda r: (r, 0, 0))],
        out_specs=[pl.BlockSpec((G, fp), lambda r: (0, 0)),
                   pl.BlockSpec((G, 8), lambda r: (0, 0))],
        out_shape=[jax.ShapeDtypeStruct((G, fp), jnp.float32),
                   jax.ShapeDtypeStruct((G, 8), jnp.float32)],
    )(*([p] * nin + [b, ids3]))


def _mlp_body(x_ref, w1_ref, b1_ref, w2_ref, b2_ref, o_ref):
    h = jnp.maximum(jnp.dot(x_ref[...], w1_ref[...],
                            preferred_element_type=jnp.float32)
                    + b1_ref[...], 0.0)
    o_ref[...] = jnp.dot(h, w2_ref[...],
                         preferred_element_type=jnp.float32) + b2_ref[...]


def _tc_mlp(x, w1, b1, w2, b2):
    return pl.pallas_call(
        _mlp_body,
        out_shape=jax.ShapeDtypeStruct((x.shape[0], w2.shape[1]),
                                       jnp.float32),
    )(x, w1, b1, w2, b2)


def _mean_mlp_body(x_ref, cnt_ref, w1_ref, b1_ref, w2_ref, b2_ref, o_ref):
    cnt = jnp.maximum(cnt_ref[:, 0:1], 1.0)
    x = x_ref[...] / cnt
    h = jnp.maximum(jnp.dot(x, w1_ref[...],
                            preferred_element_type=jnp.float32)
                    + b1_ref[...], 0.0)
    o_ref[...] = jnp.dot(h, w2_ref[...],
                         preferred_element_type=jnp.float32) + b2_ref[...]


def _tc_mean_mlp(pooled, cnt, w1, b1, w2, b2):
    return pl.pallas_call(
        _mean_mlp_body,
        out_shape=jax.ShapeDtypeStruct((G, w2.shape[1]), jnp.float32),
    )(pooled, cnt, w1, b1, w2, b2)


def _cnn_body(ids_ref, emb_ref, w1_ref, b1_ref, w2_ref, b2_ref, o_ref):
    ids = ids_ref[0]
    oh = (lax.broadcasted_iota(jnp.int32, (65, 128), 0) == ids).astype(
        jnp.float32)
    xe = lax.dot_general(oh, emb_ref[...], (((0,), (0,)), ((), ())),
                         preferred_element_type=jnp.float32)
    z1 = jnp.zeros((1, 64), jnp.float32)
    xp = jnp.concatenate([z1, xe[0:100], z1], axis=0)
    y1 = jnp.dot(xp[0:99], w1_ref[0], preferred_element_type=jnp.float32)
    for k2 in range(1, 4):
        y1 += jnp.dot(xp[k2:k2 + 99], w1_ref[k2],
                      preferred_element_type=jnp.float32)
    y1 = jnp.maximum(y1 + b1_ref[...], 0.0)
    z2 = jnp.zeros((1, 512), jnp.float32)
    y1p = jnp.concatenate([z2, y1, z2], axis=0)
    y2 = jnp.dot(y1p[0:98], w2_ref[0], preferred_element_type=jnp.float32)
    for k2 in range(1, 4):
        y2 += jnp.dot(y1p[k2:k2 + 98], w2_ref[k2],
                      preferred_element_type=jnp.float32)
    y2 = jnp.maximum(y2 + b2_ref[...], 0.0)
    o_ref[...] = jnp.max(y2[0:97], axis=0, keepdims=True)[None]


def _tc_cnn(ids3, emb, w1r, b1, w2r, b2):
    return pl.pallas_call(
        _cnn_body,
        grid=(G,),
        in_specs=[pl.BlockSpec((1, 1, 128), lambda g: (g, 0, 0)),
                  pl.BlockSpec((65, 64), lambda g: (0, 0)),
                  pl.BlockSpec((4, 64, 512), lambda g: (0, 0, 0)),
                  pl.BlockSpec((1, 512), lambda g: (0, 0)),
                  pl.BlockSpec((4, 512, 256), lambda g: (0, 0, 0)),
                  pl.BlockSpec((1, 256), lambda g: (0, 0))],
        out_specs=pl.BlockSpec((1, 1, 256), lambda g: (g, 0, 0)),
        out_shape=jax.ShapeDtypeStruct((G, 1, 256), jnp.float32),
    )(ids3, emb, w1r, b1, w2r, b2).reshape(G, 256)


def _edge_setup(edge_index):
    src = edge_index[0].astype(jnp.int32)
    dst = edge_index[1].astype(jnp.int32)
    loop = jnp.arange(N_NODES, dtype=jnp.int32)
    npad = NCH_MAX * 16 * K - 2 * N_NODES - src.shape[0] + N_NODES
    ar = jnp.arange(npad, dtype=jnp.int32)
    s_buf = jnp.concatenate([src, loop, 10016 + (ar % 224)])
    d_buf = jnp.concatenate([dst, loop, N_NODES + (ar % 240)])

    def _lay(b):
        return b.reshape(NCH_MAX, 16, K).transpose(1, 0, 2)
    return _lay(s_buf), _lay(d_buf)


def _arrange_w2(w, fs, fsp, fop):
    out = jnp.zeros((2 * fsp, fop), jnp.float32)
    out = out.at[0:fs, 0:w.shape[1]].set(w[0:fs])
    out = out.at[fsp:fsp + fs, 0:w.shape[1]].set(w[fs:2 * fs])
    return out


def _a8(p, fop):
    a = jnp.zeros((8, fop), jnp.float32)
    a = a.at[0, 0:p["a_src"].shape[0]].set(p["a_src"])
    a = a.at[1, 0:p["a_dst"].shape[0]].set(p["a_dst"])
    return a


def _gat_stack(x, edge_index, batch, supp, layers, dims, out_mlp):
    s2, d2 = _edge_setup(edge_index)
    bat3 = jnp.concatenate(
        [batch.astype(jnp.int32),
         jnp.full((NACC - N_NODES,), G, jnp.int32)]).reshape(NACC // RB, 1, RB)

    fin0 = dims[0][0]
    x_p = _pad2(x, NACC, fin0)
    p_half = None
    for i, (fin, fout) in enumerate(dims):
        pp = layers[i]
        w = _pad2(pp["W"], fin, fout)
        if i == 2:
            fs = layers[1]["W"].shape[1]
            w = _arrange_w2(pp["W"], fs, fin // 2, fout)
        a8 = _a8(pp, fout)
        if i == 0:
            h, av = _tc_layer("x", fin, fout, [x_p, w, a8])
        elif i == 2:
            bprev = _pad1(layers[1]["b"], fin // 2).reshape(1, fin // 2)
            h, av = _tc_layer("ps", fin, fout,
                              [p_half] * (fin // 2 // CB)
                              + [bprev, supp, bat3, w, a8])
        else:
            bprev = _pad1(layers[i - 1]["b"], fin).reshape(1, fin)
            h, av = _tc_layer("p", fin, fout,
                              [p_half] * (fin // CB) + [bprev, w, a8])
        p_half = _sc_gat(fout // CB, h, av[0], av[1], s2, d2)

    fp = dims[-1][1]
    b_last = _pad1(layers[-1]["b"], fp).reshape(1, fp)
    pooled, cnt = _tc_pool(fp, p_half, b_last, bat3)
    w1 = _pad2(out_mlp["l1"]["W"], fp, 1024)
    return _tc_mean_mlp(pooled, cnt, w1,
                        out_mlp["l1"]["b"].reshape(1, 1024),
                        out_mlp["l2"]["W"],
                        out_mlp["l2"]["b"].reshape(1, 128))


def kernel(drug_x, drug_edge_index, drug_batch, drug_smiles, target_x,
           target_edge_index, target_batch, target_esm2, params):
    cnn = params["cnn"]
    ids3 = jnp.pad(drug_smiles.astype(jnp.int32),
                   ((0, 0), (0, 28))).reshape(G, 1, 128)
    w1r = jnp.transpose(cnn["w1"], (2, 1, 0))
    w2r = jnp.pad(jnp.transpose(cnn["w2"], (2, 1, 0)),
                  ((0, 0), (0, 0), (0, 100)))
    drug_supp = _tc_cnn(ids3, cnn["emb"], w1r,
                        cnn["b1"].reshape(1, 512), w2r,
                        _pad1(cnn["b2"], 256).reshape(1, 256))

    esm = params["esm"]
    target_supp = _tc_mlp(target_esm2, esm["l1"]["W"],
                          esm["l1"]["b"].reshape(1, 1024),
                          _pad2(esm["l2"]["W"], 1024, 128),
                          _pad1(esm["l2"]["b"], 128).reshape(1, 128))

    d_out = _gat_stack(drug_x, drug_edge_index, drug_batch, drug_supp,
                       params["drug_gat"],
                       [(128, 128), (128, 256), (512, 384), (384, 384)],
                       params["drug_out"])
    t_out = _gat_stack(target_x, target_edge_index, target_batch, target_supp,
                       params["target_gat"],
                       [(128, 128), (128, 128), (256, 256), (256, 256)],
                       params["target_out"])
    return d_out, t_out
